# baseline (speedup 1.0000x reference)
import dataclasses

import jax
import jax.numpy as jnp
from jax import lax
from jax.experimental import pallas as pl
from jax.experimental.pallas import tpu as pltpu
from jax.experimental.pallas import tpu_sc as plsc

N = 10000
NPAD = 10240
E = 320000
F = 128
NC = 2
NS = 16
NW = NC * NS
EPW = E // NW
CHUNK = 80
NCHUNK = EPW // CHUNK
RPT = NPAD // NS
ZR = 8

CK = 128
NCHK = 80
EPT = CK * NCHK
EPAD = EPT * NW
NP2 = 10112
RP2 = NP2 // NS

_MESH = plsc.VectorSubcoreMesh(
    core_axis_name="c", subcore_axis_name="s", num_cores=NC, num_subcores=NS
)

_CP = pltpu.CompilerParams()
if "needs_layout_passes" in pltpu.CompilerParams.__dataclass_fields__:
    _CP = dataclasses.replace(_CP, needs_layout_passes=False)

_f32 = jnp.float32


def _hist_body(src_hbm, dst_hbm, do_hbm, di_hbm, sall, dall, doacc, diacc):
    cid = lax.axis_index("c")
    sid = lax.axis_index("s")
    wid = sid * NC + cid

    e0 = wid * EPW
    pltpu.sync_copy(src_hbm.at[pl.ds(e0, EPW)], sall)
    pltpu.sync_copy(dst_hbm.at[pl.ds(e0, EPW)], dall)

    @pl.loop(0, NPAD // 16)
    def _(i):
        z = jnp.zeros((16,), _f32)
        doacc[pl.ds(i * 16, 16)] = z
        diacc[pl.ds(i * 16, 16)] = z

    @pl.loop(0, EPW // 16)
    def _(r):
        si = sall[pl.ds(r * 16, 16)]
        di = dall[pl.ds(r * 16, 16)]
        one = jnp.ones((16,), _f32)
        plsc.addupdate_scatter(doacc, [si], one)
        plsc.addupdate_scatter(diacc, [di], one)

    pltpu.sync_copy(doacc, do_hbm.at[wid])
    pltpu.sync_copy(diacc, di_hbm.at[wid])


_hist = pl.kernel(
    _hist_body,
    out_type=(
        jax.ShapeDtypeStruct((NW, NPAD), _f32),
        jax.ShapeDtypeStruct((NW, NPAD), _f32),
    ),
    mesh=_MESH,
    compiler_params=_CP,
    scratch_types=[
        pltpu.VMEM((EPW,), jnp.int32),
        pltpu.VMEM((EPW,), jnp.int32),
        pltpu.VMEM((NPAD,), _f32),
        pltpu.VMEM((NPAD,), _f32),
    ],
)


def _wk_body(src_hbm, dst_hbm, nd_hbm, w_hbm, sall, dall, ndtab, wacc):
    cid = lax.axis_index("c")
    sid = lax.axis_index("s")
    wid = sid * NC + cid

    e0 = wid * EPW
    pltpu.sync_copy(src_hbm.at[pl.ds(e0, EPW)], sall)
    pltpu.sync_copy(dst_hbm.at[pl.ds(e0, EPW)], dall)
    pltpu.sync_copy(nd_hbm, ndtab)

    @pl.loop(0, NPAD // 16)
    def _(i):
        wacc[pl.ds(i * 16, 16)] = jnp.zeros((16,), _f32)

    @pl.loop(0, EPW // 16)
    def _(r):
        si = sall[pl.ds(r * 16, 16)]
        di = dall[pl.ds(r * 16, 16)]
        vals = plsc.load_gather(ndtab, [di])
        plsc.addupdate_scatter(wacc, [si], vals)

    pltpu.sync_copy(wacc, w_hbm.at[wid])


_wk = pl.kernel(
    _wk_body,
    out_type=jax.ShapeDtypeStruct((NW, NPAD), _f32),
    mesh=_MESH,
    compiler_params=_CP,
    scratch_types=[
        pltpu.VMEM((EPW,), jnp.int32),
        pltpu.VMEM((EPW,), jnp.int32),
        pltpu.VMEM((NPAD,), _f32),
        pltpu.VMEM((NPAD,), _f32),
    ],
)


def _agg_body(table_hbm, src_hbm, dst_hbm, agg_hbm,
              sall, dall, sidx0, sidx1, didx0, didx1,
              rows0, rows1, zb, acc,
              semg0, semg1, semi):
    sidx = (sidx0, sidx1)
    didx = (didx0, didx1)
    rows = (rows0, rows1)
    semg = (semg0, semg1)

    cid = lax.axis_index("c")
    sid = lax.axis_index("s")
    wid = sid * NC + cid

    e0 = wid * (EPT // 2)
    pltpu.async_copy(src_hbm.at[pl.ds(e0, EPT // 2)], sall, semi)
    pltpu.async_copy(dst_hbm.at[pl.ds(e0, EPT // 2)], dall, semi)

    @pl.loop(0, ZR)
    def _(i):
        @pl.loop(0, F // 16)
        def _(j):
            zb[i, pl.ds(j * 16, 16)] = jnp.zeros((16,), _f32)

    a0 = sid * RP2

    @pl.loop(0, RP2 // ZR)
    def _(k):
        pltpu.sync_copy(zb, acc.at[pl.ds(a0 + k * ZR, ZR)])

    pltpu.make_async_copy(src_hbm.at[pl.ds(e0, EPT // 2)], sall, semi).wait()
    pltpu.make_async_copy(dst_hbm.at[pl.ds(e0, EPT // 2)], dall, semi).wait()
    plsc.subcore_barrier()

    def rcopy(g, b):
        for k in range(CK // 32):
            sv = sall[pl.ds(g * (CK // 2) + k * 16, 16)]
            sidx[b][pl.ds(k * 32, 16)] = sv & 0xFFFF
            sidx[b][pl.ds(k * 32 + 16, 16)] = lax.shift_right_logical(sv, 16)
            dv = dall[pl.ds(g * (CK // 2) + k * 16, 16)]
            didx[b][pl.ds(k * 32, 16)] = dv & 0xFFFF
            didx[b][pl.ds(k * 32 + 16, 16)] = lax.shift_right_logical(dv, 16)

    def gstart(b):
        pltpu.async_copy(table_hbm.at[sidx[b]], rows[b], semg[b])

    def gwait(b):
        pltpu.make_async_copy(table_hbm.at[sidx[b]], rows[b], semg[b]).wait()

    def scat(b):
        pltpu.sync_copy(rows[b], acc.at[didx[b]], add=True)

    rcopy(0, 0)
    gstart(0)

    @pl.loop(0, (NCHK - 2) // 2)
    def _(it):
        g = 2 * it
        rcopy(g + 1, 1)
        gstart(1)
        gwait(0)
        scat(0)
        rcopy(g + 2, 0)
        gstart(0)
        gwait(1)
        scat(1)

    rcopy(NCHK - 1, 1)
    gstart(1)
    gwait(0)
    scat(0)
    gwait(1)
    scat(1)

    plsc.subcore_barrier()

    pltpu.sync_copy(acc.at[pl.ds(a0, RP2)], agg_hbm.at[cid, pl.ds(a0, RP2)])


_agg = pl.kernel(
    _agg_body,
    out_type=jax.ShapeDtypeStruct((NC, NP2, F), _f32),
    mesh=_MESH,
    compiler_params=_CP,
    scratch_types=[
        pltpu.VMEM((EPT // 2,), jnp.int32),
        pltpu.VMEM((EPT // 2,), jnp.int32),
        pltpu.VMEM((CK,), jnp.int32),
        pltpu.VMEM((CK,), jnp.int32),
        pltpu.VMEM((CK,), jnp.int32),
        pltpu.VMEM((CK,), jnp.int32),
        pltpu.VMEM((CK, F), _f32),
        pltpu.VMEM((CK, F), _f32),
        pltpu.VMEM((ZR, F), _f32),
        pltpu.VMEM_SHARED((NP2, F), _f32),
        pltpu.SemaphoreType.DMA,
        pltpu.SemaphoreType.DMA,
        pltpu.SemaphoreType.DMA,
    ],
)


def _prep_body(x_ref, do_ref, di_ref, xn_ref, nd1_ref, ndb_ref, nob_ref):
    deg_o = jnp.sum(do_ref[...], axis=0)
    deg_i = jnp.sum(di_ref[...], axis=0)
    no = lax.rsqrt(jnp.clip(deg_o, 1.0, None))
    nd = lax.rsqrt(jnp.clip(deg_i, 1.0, None))
    nd1_ref[...] = nd
    ndb_ref[...] = nd[:, None] * jnp.ones((1, 16), _f32)
    nob_ref[...] = no[:, None] * jnp.ones((1, 16), _f32)
    xn_ref[:N, :] = x_ref[...] * no[:N, None]
    xn_ref[N:, :] = jnp.zeros((NP2 - N, F), _f32)


def _mid_body(a_ref, ndb_ref, nob_ref, w1_ref, b1_ref, w2_ref, m2_ref):
    aggv = a_ref[0, :N, :] + a_ref[1, :N, :]
    h = jnp.dot(aggv, w1_ref[...], preferred_element_type=_f32)
    h = jax.nn.relu(h * ndb_ref[:N, 0:1] + b1_ref[...][None, :])
    h = h * nob_ref[:N, 0:1]
    m2_ref[:N, :] = jnp.dot(h, w2_ref[...], preferred_element_type=_f32)
    m2_ref[N:, :] = jnp.zeros((NP2 - N, F), _f32)


def _fin_body(a_ref, ndb_ref, nob_ref, w_ref, b2_ref, w3_ref, b3_ref, o_ref):
    aggv = a_ref[0, :N, :] + a_ref[1, :N, :]
    h2 = jax.nn.relu(aggv * ndb_ref[:N, 0:1] + b2_ref[...][None, :])
    w = jnp.sum(w_ref[...], axis=0)
    wn = w[:N, None] * nob_ref[:N, 0:1]
    u = jnp.sum(h2 * wn, axis=0)
    logits = (jnp.dot(u[None, :], w3_ref[...], preferred_element_type=_f32)
              * (1.0 / N) + b3_ref[...][None, :])
    m = jnp.max(logits, axis=1, keepdims=True)
    ex = jnp.exp(logits - m)
    o_ref[...] = ex / jnp.sum(ex, axis=1, keepdims=True)


_prep = pl.pallas_call(
    _prep_body,
    out_shape=(
        jax.ShapeDtypeStruct((NP2, F), _f32),
        jax.ShapeDtypeStruct((NPAD,), _f32),
        jax.ShapeDtypeStruct((NPAD, 16), _f32),
        jax.ShapeDtypeStruct((NPAD, 16), _f32),
    ),
)

_mid = pl.pallas_call(
    _mid_body,
    out_shape=jax.ShapeDtypeStruct((NP2, F), _f32),
)

_fin = pl.pallas_call(
    _fin_body,
    out_shape=jax.ShapeDtypeStruct((1, 16), _f32),
)


@jax.jit
def _run(in_feat, edge_index, W1, b1, W2, b2, W3, b3):
    src = edge_index[0].astype(jnp.int32)
    dst = edge_index[1].astype(jnp.int32)
    pad = jnp.full((EPAD - E,), N, jnp.int32)
    srcp = jnp.concatenate([src, pad])
    dstp = jnp.concatenate([dst, pad])
    src16 = srcp[0::2] | (srcp[1::2] << 16)
    dst16 = dstp[0::2] | (dstp[1::2] << 16)
    deg_o, deg_i = _hist(src, dst)
    xn, nd1, ndb, nob = _prep(in_feat, deg_o, deg_i)
    agg1 = _agg(xn, src16, dst16)
    w = _wk(src, dst, nd1)
    m2 = _mid(agg1, ndb, nob, W1, b1, W2)
    agg2 = _agg(m2, src16, dst16)
    return _fin(agg2, ndb, nob, w, b2, W3, b3)


def kernel(in_feat, edge_index, W1, b1, W2, b2, W3, b3, shape):
    return _run(in_feat, edge_index, W1, b1, W2, b2, W3, b3)


# speedup vs baseline: 3.1721x; 3.1721x over previous
import dataclasses

import jax
import jax.numpy as jnp
from jax import lax
from jax.experimental import pallas as pl
from jax.experimental.pallas import tpu as pltpu
from jax.experimental.pallas import tpu_sc as plsc

N = 10000
NPAD = 10240
E = 320000
F = 128
NC = 2
NS = 16
NW = NC * NS
EPW = E // NW
CHUNK = 80
NCHUNK = EPW // CHUNK
RPT = NPAD // NS
ZR = 8

CK = 128
NCHK = 80
EPT = CK * NCHK
EPAD = EPT * NW
NP2 = 10112
RP2 = NP2 // NS

_MESH = plsc.VectorSubcoreMesh(
    core_axis_name="c", subcore_axis_name="s", num_cores=NC, num_subcores=NS
)

_CP = pltpu.CompilerParams()
if "needs_layout_passes" in pltpu.CompilerParams.__dataclass_fields__:
    _CP = dataclasses.replace(_CP, needs_layout_passes=False)

_f32 = jnp.float32


def _hist_body(src_hbm, dst_hbm, do_hbm, di_hbm, sall, dall, doacc, diacc):
    cid = lax.axis_index("c")
    sid = lax.axis_index("s")
    wid = sid * NC + cid

    e0 = wid * EPW
    pltpu.sync_copy(src_hbm.at[pl.ds(e0, EPW)], sall)
    pltpu.sync_copy(dst_hbm.at[pl.ds(e0, EPW)], dall)

    @pl.loop(0, NPAD // 16)
    def _(i):
        z = jnp.zeros((16,), _f32)
        doacc[pl.ds(i * 16, 16)] = z
        diacc[pl.ds(i * 16, 16)] = z

    @pl.loop(0, EPW // 16)
    def _(r):
        si = sall[pl.ds(r * 16, 16)]
        di = dall[pl.ds(r * 16, 16)]
        one = jnp.ones((16,), _f32)
        plsc.addupdate_scatter(doacc, [si], one)
        plsc.addupdate_scatter(diacc, [di], one)

    pltpu.sync_copy(doacc, do_hbm.at[wid])
    pltpu.sync_copy(diacc, di_hbm.at[wid])


_hist = pl.kernel(
    _hist_body,
    out_type=(
        jax.ShapeDtypeStruct((NW, NPAD), _f32),
        jax.ShapeDtypeStruct((NW, NPAD), _f32),
    ),
    mesh=_MESH,
    compiler_params=_CP,
    scratch_types=[
        pltpu.VMEM((EPW,), jnp.int32),
        pltpu.VMEM((EPW,), jnp.int32),
        pltpu.VMEM((NPAD,), _f32),
        pltpu.VMEM((NPAD,), _f32),
    ],
)


def _wk_body(src_hbm, dst_hbm, nd_hbm, w_hbm, sall, dall, ndtab, wacc):
    cid = lax.axis_index("c")
    sid = lax.axis_index("s")
    wid = sid * NC + cid

    e0 = wid * EPW
    pltpu.sync_copy(src_hbm.at[pl.ds(e0, EPW)], sall)
    pltpu.sync_copy(dst_hbm.at[pl.ds(e0, EPW)], dall)
    pltpu.sync_copy(nd_hbm, ndtab)

    @pl.loop(0, NPAD // 16)
    def _(i):
        wacc[pl.ds(i * 16, 16)] = jnp.zeros((16,), _f32)

    @pl.loop(0, EPW // 16)
    def _(r):
        si = sall[pl.ds(r * 16, 16)]
        di = dall[pl.ds(r * 16, 16)]
        vals = plsc.load_gather(ndtab, [di])
        plsc.addupdate_scatter(wacc, [si], vals)

    pltpu.sync_copy(wacc, w_hbm.at[wid])


_wk = pl.kernel(
    _wk_body,
    out_type=jax.ShapeDtypeStruct((NW, NPAD), _f32),
    mesh=_MESH,
    compiler_params=_CP,
    scratch_types=[
        pltpu.VMEM((EPW,), jnp.int32),
        pltpu.VMEM((EPW,), jnp.int32),
        pltpu.VMEM((NPAD,), _f32),
        pltpu.VMEM((NPAD,), _f32),
    ],
)


def _agg_body(table_hbm, src_hbm, dst_hbm, agg_hbm,
              sall, dall, sidx0, sidx1, didx0, didx1,
              rows0, rows1, zb, acc,
              semg0, semg1, semi):
    sidx = (sidx0, sidx1)
    didx = (didx0, didx1)
    rows = (rows0, rows1)
    semg = (semg0, semg1)

    cid = lax.axis_index("c")
    sid = lax.axis_index("s")
    wid = sid * NC + cid

    e0 = wid * (EPT // 2)
    pltpu.async_copy(src_hbm.at[pl.ds(e0, EPT // 2)], sall, semi)
    pltpu.async_copy(dst_hbm.at[pl.ds(e0, EPT // 2)], dall, semi)

    @pl.loop(0, ZR)
    def _(i):
        @pl.loop(0, F // 16)
        def _(j):
            zb[i, pl.ds(j * 16, 16)] = jnp.zeros((16,), _f32)

    a0 = sid * RP2

    @pl.loop(0, RP2 // ZR)
    def _(k):
        pltpu.sync_copy(zb, acc.at[pl.ds(a0 + k * ZR, ZR)])

    pltpu.make_async_copy(src_hbm.at[pl.ds(e0, EPT // 2)], sall, semi).wait()
    pltpu.make_async_copy(dst_hbm.at[pl.ds(e0, EPT // 2)], dall, semi).wait()
    plsc.subcore_barrier()

    def rcopy(g, b):
        for k in range(CK // 32):
            sv = sall[pl.ds(g * (CK // 2) + k * 16, 16)]
            sidx[b][pl.ds(k * 32, 16)] = sv & 0xFFFF
            sidx[b][pl.ds(k * 32 + 16, 16)] = lax.shift_right_logical(sv, 16)
            dv = dall[pl.ds(g * (CK // 2) + k * 16, 16)]
            didx[b][pl.ds(k * 32, 16)] = dv & 0xFFFF
            didx[b][pl.ds(k * 32 + 16, 16)] = lax.shift_right_logical(dv, 16)

    def gstart(b):
        pltpu.async_copy(table_hbm.at[sidx[b]], rows[b], semg[b])

    def gwait(b):
        pltpu.make_async_copy(table_hbm.at[sidx[b]], rows[b], semg[b]).wait()

    def scat(b):
        pltpu.sync_copy(rows[b], acc.at[didx[b]], add=True)

    rcopy(0, 0)
    gstart(0)

    @pl.loop(0, (NCHK - 2) // 2)
    def _(it):
        g = 2 * it
        rcopy(g + 1, 1)
        gstart(1)
        gwait(0)
        scat(0)
        rcopy(g + 2, 0)
        gstart(0)
        gwait(1)
        scat(1)

    rcopy(NCHK - 1, 1)
    gstart(1)
    gwait(0)
    scat(0)
    gwait(1)
    scat(1)

    plsc.subcore_barrier()

    pltpu.sync_copy(acc.at[pl.ds(a0, RP2)], agg_hbm.at[cid, pl.ds(a0, RP2)])


_agg = pl.kernel(
    _agg_body,
    out_type=jax.ShapeDtypeStruct((NC, NP2, F), _f32),
    mesh=_MESH,
    compiler_params=_CP,
    scratch_types=[
        pltpu.VMEM((EPT // 2,), jnp.int32),
        pltpu.VMEM((EPT // 2,), jnp.int32),
        pltpu.VMEM((CK,), jnp.int32),
        pltpu.VMEM((CK,), jnp.int32),
        pltpu.VMEM((CK,), jnp.int32),
        pltpu.VMEM((CK,), jnp.int32),
        pltpu.VMEM((CK, F), _f32),
        pltpu.VMEM((CK, F), _f32),
        pltpu.VMEM((ZR, F), _f32),
        pltpu.VMEM_SHARED((NP2, F), _f32),
        pltpu.SemaphoreType.DMA,
        pltpu.SemaphoreType.DMA,
        pltpu.SemaphoreType.DMA,
    ],
)


def _prep_body(x_ref, do_ref, di_ref, xn_ref, nd1_ref, ndb_ref, nob_ref):
    deg_o = jnp.sum(do_ref[...], axis=0)
    deg_i = jnp.sum(di_ref[...], axis=0)
    no = lax.rsqrt(jnp.clip(deg_o, 1.0, None))
    nd = lax.rsqrt(jnp.clip(deg_i, 1.0, None))
    nd1_ref[...] = nd
    ndb_ref[...] = nd[:, None] * jnp.ones((1, 16), _f32)
    nob_ref[...] = no[:, None] * jnp.ones((1, 16), _f32)
    xn_ref[:N, :] = x_ref[...] * no[:N, None]
    xn_ref[N:, :] = jnp.zeros((NP2 - N, F), _f32)


def _mid_body(a_ref, ndb_ref, nob_ref, w1_ref, b1_ref, w2_ref, m2_ref):
    aggv = a_ref[0, :N, :] + a_ref[1, :N, :]
    h = jnp.dot(aggv, w1_ref[...], preferred_element_type=_f32)
    h = jax.nn.relu(h * ndb_ref[:N, 0:1] + b1_ref[...][None, :])
    h = h * nob_ref[:N, 0:1]
    m2_ref[:N, :] = jnp.dot(h, w2_ref[...], preferred_element_type=_f32)
    m2_ref[N:, :] = jnp.zeros((NP2 - N, F), _f32)


def _fin_body(a_ref, ndb_ref, nob_ref, w_ref, b2_ref, w3_ref, b3_ref, o_ref):
    aggv = a_ref[0, :N, :] + a_ref[1, :N, :]
    h2 = jax.nn.relu(aggv * ndb_ref[:N, 0:1] + b2_ref[...][None, :])
    w = jnp.sum(w_ref[...], axis=0)
    wn = w[:N, None] * nob_ref[:N, 0:1]
    u = jnp.sum(h2 * wn, axis=0)
    logits = (jnp.dot(u[None, :], w3_ref[...], preferred_element_type=_f32)
              * (1.0 / N) + b3_ref[...][None, :])
    m = jnp.max(logits, axis=1, keepdims=True)
    ex = jnp.exp(logits - m)
    o_ref[...] = ex / jnp.sum(ex, axis=1, keepdims=True)


_prep = pl.pallas_call(
    _prep_body,
    out_shape=(
        jax.ShapeDtypeStruct((NP2, F), _f32),
        jax.ShapeDtypeStruct((NPAD,), _f32),
        jax.ShapeDtypeStruct((NPAD, 16), _f32),
        jax.ShapeDtypeStruct((NPAD, 16), _f32),
    ),
)

_mid = pl.pallas_call(
    _mid_body,
    out_shape=jax.ShapeDtypeStruct((NP2, F), _f32),
)

_fin = pl.pallas_call(
    _fin_body,
    out_shape=jax.ShapeDtypeStruct((1, 16), _f32),
)


@jax.jit
def _run(in_feat, edge_index, W1, b1, W2, b2, W3, b3):
    src = edge_index[0].astype(jnp.int32)
    dst = edge_index[1].astype(jnp.int32)
    pad = N + (jnp.arange(EPAD - E, dtype=jnp.int32) % (NP2 - N))
    srcp = jnp.concatenate([src, pad])
    dstp = jnp.concatenate([dst, pad])
    src16 = srcp[0::2] | (srcp[1::2] << 16)
    dst16 = dstp[0::2] | (dstp[1::2] << 16)
    deg_o, deg_i = _hist(src, dst)
    xn, nd1, ndb, nob = _prep(in_feat, deg_o, deg_i)
    agg1 = _agg(xn, src16, dst16)
    w = _wk(src, dst, nd1)
    m2 = _mid(agg1, ndb, nob, W1, b1, W2)
    agg2 = _agg(m2, src16, dst16)
    return _fin(agg2, ndb, nob, w, b2, W3, b3)


def kernel(in_feat, edge_index, W1, b1, W2, b2, W3, b3, shape):
    return _run(in_feat, edge_index, W1, b1, W2, b2, W3, b3)


# speedup vs baseline: 4.1518x; 1.3088x over previous
import dataclasses

import jax
import jax.numpy as jnp
from jax import lax
from jax.experimental import pallas as pl
from jax.experimental.pallas import tpu as pltpu
from jax.experimental.pallas import tpu_sc as plsc

N = 10000
NPAD = 10240
E = 320000
F = 128
NC = 2
NS = 16
NW = NC * NS
EPW = E // NW
CHUNK = 80
NCHUNK = EPW // CHUNK
RPT = NPAD // NS
ZR = 8

CK = 128
NCHK = 80
EPT = CK * NCHK
EPAD = EPT * NW
NP2 = 10112
RP2 = NP2 // NS

_MESH = plsc.VectorSubcoreMesh(
    core_axis_name="c", subcore_axis_name="s", num_cores=NC, num_subcores=NS
)

_CP = pltpu.CompilerParams()
if "needs_layout_passes" in pltpu.CompilerParams.__dataclass_fields__:
    _CP = dataclasses.replace(_CP, needs_layout_passes=False)

_f32 = jnp.float32


def _hist_body(src_hbm, dst_hbm, do_hbm, di_hbm, sall, dall, doacc, diacc):
    cid = lax.axis_index("c")
    sid = lax.axis_index("s")
    wid = sid * NC + cid

    e0 = wid * EPW
    pltpu.sync_copy(src_hbm.at[pl.ds(e0, EPW)], sall)
    pltpu.sync_copy(dst_hbm.at[pl.ds(e0, EPW)], dall)

    @pl.loop(0, NPAD // 16)
    def _(i):
        z = jnp.zeros((16,), _f32)
        doacc[pl.ds(i * 16, 16)] = z
        diacc[pl.ds(i * 16, 16)] = z

    @pl.loop(0, EPW // 16)
    def _(r):
        si = sall[pl.ds(r * 16, 16)]
        di = dall[pl.ds(r * 16, 16)]
        one = jnp.ones((16,), _f32)
        plsc.addupdate_scatter(doacc, [si], one)
        plsc.addupdate_scatter(diacc, [di], one)

    pltpu.sync_copy(doacc, do_hbm.at[wid])
    pltpu.sync_copy(diacc, di_hbm.at[wid])


_hist = pl.kernel(
    _hist_body,
    out_type=(
        jax.ShapeDtypeStruct((NW, NPAD), _f32),
        jax.ShapeDtypeStruct((NW, NPAD), _f32),
    ),
    mesh=_MESH,
    compiler_params=_CP,
    scratch_types=[
        pltpu.VMEM((EPW,), jnp.int32),
        pltpu.VMEM((EPW,), jnp.int32),
        pltpu.VMEM((NPAD,), _f32),
        pltpu.VMEM((NPAD,), _f32),
    ],
)


def _wk_body(src_hbm, dst_hbm, nd_hbm, w_hbm, sall, dall, ndtab, wacc):
    cid = lax.axis_index("c")
    sid = lax.axis_index("s")
    wid = sid * NC + cid

    e0 = wid * EPW
    pltpu.sync_copy(src_hbm.at[pl.ds(e0, EPW)], sall)
    pltpu.sync_copy(dst_hbm.at[pl.ds(e0, EPW)], dall)
    pltpu.sync_copy(nd_hbm, ndtab)

    @pl.loop(0, NPAD // 16)
    def _(i):
        wacc[pl.ds(i * 16, 16)] = jnp.zeros((16,), _f32)

    @pl.loop(0, EPW // 16)
    def _(r):
        si = sall[pl.ds(r * 16, 16)]
        di = dall[pl.ds(r * 16, 16)]
        vals = plsc.load_gather(ndtab, [di])
        plsc.addupdate_scatter(wacc, [si], vals)

    pltpu.sync_copy(wacc, w_hbm.at[wid])


_wk = pl.kernel(
    _wk_body,
    out_type=jax.ShapeDtypeStruct((NW, NPAD), _f32),
    mesh=_MESH,
    compiler_params=_CP,
    scratch_types=[
        pltpu.VMEM((EPW,), jnp.int32),
        pltpu.VMEM((EPW,), jnp.int32),
        pltpu.VMEM((NPAD,), _f32),
        pltpu.VMEM((NPAD,), _f32),
    ],
)


def _agg_body(table_hbm, src_hbm, dst_hbm, agg_hbm,
              sall, dall, sidx0, sidx1, didx0, didx1,
              rows0, rows1, zb, acc,
              semg0, semg1, semi):
    sidx = (sidx0, sidx1)
    didx = (didx0, didx1)
    rows = (rows0, rows1)
    semg = (semg0, semg1)

    cid = lax.axis_index("c")
    sid = lax.axis_index("s")
    wid = sid * NC + cid

    e0 = wid * (EPT // 2)
    pltpu.async_copy(src_hbm.at[pl.ds(e0, EPT // 2)], sall, semi)
    pltpu.async_copy(dst_hbm.at[pl.ds(e0, EPT // 2)], dall, semi)

    @pl.loop(0, ZR)
    def _(i):
        @pl.loop(0, F // 16)
        def _(j):
            zb[i, pl.ds(j * 16, 16)] = jnp.zeros((16,), _f32)

    a0 = sid * RP2

    @pl.loop(0, RP2 // ZR)
    def _(k):
        pltpu.sync_copy(zb, acc.at[pl.ds(a0 + k * ZR, ZR)])

    pltpu.make_async_copy(src_hbm.at[pl.ds(e0, EPT // 2)], sall, semi).wait()
    pltpu.make_async_copy(dst_hbm.at[pl.ds(e0, EPT // 2)], dall, semi).wait()
    plsc.subcore_barrier()

    def rcopy(g, b):
        for k in range(CK // 32):
            sv = sall[pl.ds(g * (CK // 2) + k * 16, 16)]
            sidx[b][pl.ds(k * 32, 16)] = sv & 0xFFFF
            sidx[b][pl.ds(k * 32 + 16, 16)] = lax.shift_right_logical(sv, 16)
            dv = dall[pl.ds(g * (CK // 2) + k * 16, 16)]
            didx[b][pl.ds(k * 32, 16)] = dv & 0xFFFF
            didx[b][pl.ds(k * 32 + 16, 16)] = lax.shift_right_logical(dv, 16)

    def gstart(b):
        pltpu.async_copy(table_hbm.at[sidx[b]], rows[b], semg[b])

    def gwait(b):
        pltpu.make_async_copy(table_hbm.at[sidx[b]], rows[b], semg[b]).wait()

    def scat(b):
        pltpu.sync_copy(rows[b], acc.at[didx[b]], add=True)

    rcopy(0, 0)
    gstart(0)

    @pl.loop(0, (NCHK - 2) // 2)
    def _(it):
        g = 2 * it
        rcopy(g + 1, 1)
        gstart(1)
        gwait(0)
        scat(0)
        rcopy(g + 2, 0)
        gstart(0)
        gwait(1)
        scat(1)

    rcopy(NCHK - 1, 1)
    gstart(1)
    gwait(0)
    scat(0)
    gwait(1)
    scat(1)

    plsc.subcore_barrier()

    pltpu.sync_copy(acc.at[pl.ds(a0, RP2)], agg_hbm.at[cid, pl.ds(a0, RP2)])


_agg = pl.kernel(
    _agg_body,
    out_type=jax.ShapeDtypeStruct((NC, NP2, F), _f32),
    mesh=_MESH,
    compiler_params=_CP,
    scratch_types=[
        pltpu.VMEM((EPT // 2,), jnp.int32),
        pltpu.VMEM((EPT // 2,), jnp.int32),
        pltpu.VMEM((CK,), jnp.int32),
        pltpu.VMEM((CK,), jnp.int32),
        pltpu.VMEM((CK,), jnp.int32),
        pltpu.VMEM((CK,), jnp.int32),
        pltpu.VMEM((CK, F), _f32),
        pltpu.VMEM((CK, F), _f32),
        pltpu.VMEM((ZR, F), _f32),
        pltpu.VMEM_SHARED((NP2, F), _f32),
        pltpu.SemaphoreType.DMA,
        pltpu.SemaphoreType.DMA,
        pltpu.SemaphoreType.DMA,
    ],
)


def _prep_body(x_ref, do_ref, di_ref, xn_ref, nd1_ref, ndb_ref, nob_ref):
    deg_o = jnp.sum(do_ref[...], axis=0)
    deg_i = jnp.sum(di_ref[...], axis=0)
    no = lax.rsqrt(jnp.clip(deg_o, 1.0, None))
    nd = lax.rsqrt(jnp.clip(deg_i, 1.0, None))
    nd1_ref[...] = nd
    ndb_ref[...] = nd[:, None] * jnp.ones((1, 16), _f32)
    nob_ref[...] = no[:, None] * jnp.ones((1, 16), _f32)
    xn_ref[:N, :] = x_ref[...] * no[:N, None]
    xn_ref[N:, :] = jnp.zeros((NP2 - N, F), _f32)


def _mid_body(a_ref, ndb_ref, nob_ref, w1_ref, b1_ref, w2_ref, m2_ref):
    aggv = a_ref[0, :N, :] + a_ref[1, :N, :]
    h = jnp.dot(aggv, w1_ref[...], preferred_element_type=_f32)
    h = jax.nn.relu(h * ndb_ref[:N, 0:1] + b1_ref[...][None, :])
    h = h * nob_ref[:N, 0:1]
    m2_ref[:N, :] = jnp.dot(h, w2_ref[...], preferred_element_type=_f32)
    m2_ref[N:, :] = jnp.zeros((NP2 - N, F), _f32)


def _fin_body(a_ref, ndb_ref, nob_ref, w_ref, b2_ref, w3_ref, b3_ref, o_ref):
    aggv = a_ref[0, :N, :] + a_ref[1, :N, :]
    h2 = jax.nn.relu(aggv * ndb_ref[:N, 0:1] + b2_ref[...][None, :])
    w = jnp.sum(w_ref[...], axis=0)
    wn = w[:N, None] * nob_ref[:N, 0:1]
    u = jnp.sum(h2 * wn, axis=0)
    logits = (jnp.dot(u[None, :], w3_ref[...], preferred_element_type=_f32)
              * (1.0 / N) + b3_ref[...][None, :])
    m = jnp.max(logits, axis=1, keepdims=True)
    ex = jnp.exp(logits - m)
    o_ref[...] = ex / jnp.sum(ex, axis=1, keepdims=True)


_prep = pl.pallas_call(
    _prep_body,
    out_shape=(
        jax.ShapeDtypeStruct((NP2, F), _f32),
        jax.ShapeDtypeStruct((NPAD,), _f32),
        jax.ShapeDtypeStruct((NPAD, 16), _f32),
        jax.ShapeDtypeStruct((NPAD, 16), _f32),
    ),
)

_mid = pl.pallas_call(
    _mid_body,
    out_shape=jax.ShapeDtypeStruct((NP2, F), _f32),
)

_fin = pl.pallas_call(
    _fin_body,
    out_shape=jax.ShapeDtypeStruct((1, 16), _f32),
)


@jax.jit
def _run(in_feat, edge_index, W1, b1, W2, b2, W3, b3):
    src = edge_index[0].astype(jnp.int32)
    dst = edge_index[1].astype(jnp.int32)
    pad = N + (jnp.arange(EPAD - E, dtype=jnp.int32) % (NP2 - N))
    srcp = jnp.concatenate([src, pad])
    dstp = jnp.concatenate([dst, pad])
    src16 = srcp[:EPAD // 2] | (srcp[EPAD // 2:] << 16)
    dst16 = dstp[:EPAD // 2] | (dstp[EPAD // 2:] << 16)
    deg_o, deg_i = _hist(src, dst)
    xn, nd1, ndb, nob = _prep(in_feat, deg_o, deg_i)
    agg1 = _agg(xn, src16, dst16)
    w = _wk(src, dst, nd1)
    m2 = _mid(agg1, ndb, nob, W1, b1, W2)
    agg2 = _agg(m2, src16, dst16)
    return _fin(agg2, ndb, nob, w, b2, W3, b3)


def kernel(in_feat, edge_index, W1, b1, W2, b2, W3, b3, shape):
    return _run(in_feat, edge_index, W1, b1, W2, b2, W3, b3)
